# Initial kernel scaffold; baseline (speedup 1.0000x reference)
import jax
import jax.numpy as jnp
from jax.experimental import pallas as pl


def kernel(buf, dWt, t):
    raise NotImplementedError("write your pallas kernel here")


# speedup vs baseline: 1.0904x; 1.0904x over previous
import functools

import jax
import jax.numpy as jnp
from jax.experimental import pallas as pl
from jax.experimental.pallas import tpu as pltpu

NH = 512
DT = 1.0
N_NODES = 100000
N_ROWS = NH + 2

BLOCK_W = 2048


def _step_kernel(t_ref, buf_ref, w_ref, out_ref, nx_ref):
    tt = t_ref[0, 0]
    out_ref[...] = buf_ref[...]
    x = buf_ref[pl.ds(NH + tt, 1), :]
    r0 = buf_ref[pl.ds(tt, 1), :]
    r1 = buf_ref[pl.ds(tt + 1, 1), :]
    w = w_ref[...]
    d1 = 0.1 * (r0 - x)
    xi = jnp.tanh(x + DT * d1 + w)
    d2 = 0.1 * (r1 - xi)
    nx = jnp.tanh(x + DT * 0.5 * (d1 + d2) + w)
    out_ref[pl.ds(NH + tt + 1, 1), :] = nx
    nx_ref[...] = nx


@functools.partial(jax.jit, static_argnames=())
def kernel(buf, dWt, t):
    w2d = dWt.reshape(1, N_NODES)
    grid = (pl.cdiv(N_NODES, BLOCK_W),)
    out_buf, nx2d = pl.pallas_call(
        _step_kernel,
        grid=grid,
        in_specs=[
            pl.BlockSpec(memory_space=pltpu.SMEM),
            pl.BlockSpec((N_ROWS, BLOCK_W), lambda j: (0, j)),
            pl.BlockSpec((1, BLOCK_W), lambda j: (0, j)),
        ],
        out_specs=[
            pl.BlockSpec((N_ROWS, BLOCK_W), lambda j: (0, j)),
            pl.BlockSpec((1, BLOCK_W), lambda j: (0, j)),
        ],
        out_shape=[
            jax.ShapeDtypeStruct((N_ROWS, N_NODES), jnp.float32),
            jax.ShapeDtypeStruct((1, N_NODES), jnp.float32),
        ],
    )(t, buf, w2d)
    return (out_buf, nx2d.reshape(N_NODES))


# speedup vs baseline: 1.1159x; 1.0234x over previous
import functools

import jax
import jax.numpy as jnp
from jax.experimental import pallas as pl
from jax.experimental.pallas import tpu as pltpu

NH = 512
DT = 1.0
N_NODES = 100000
N_ROWS = NH + 2

BLOCK_W = 4096


def _step_kernel(t_ref, buf_ref, w_ref, out_ref, nx_ref):
    tt = t_ref[0, 0]
    out_ref[...] = buf_ref[...]
    x = buf_ref[pl.ds(NH + tt, 1), :]
    r0 = buf_ref[pl.ds(tt, 1), :]
    r1 = buf_ref[pl.ds(tt + 1, 1), :]
    w = w_ref[...]
    d1 = 0.1 * (r0 - x)
    xi = jnp.tanh(x + DT * d1 + w)
    d2 = 0.1 * (r1 - xi)
    nx = jnp.tanh(x + DT * 0.5 * (d1 + d2) + w)
    out_ref[pl.ds(NH + tt + 1, 1), :] = nx
    nx_ref[...] = nx


@functools.partial(jax.jit, static_argnames=())
def kernel(buf, dWt, t):
    w2d = dWt.reshape(1, N_NODES)
    grid = (pl.cdiv(N_NODES, BLOCK_W),)
    out_buf, nx2d = pl.pallas_call(
        _step_kernel,
        grid=grid,
        in_specs=[
            pl.BlockSpec(memory_space=pltpu.SMEM),
            pl.BlockSpec((N_ROWS, BLOCK_W), lambda j: (0, j)),
            pl.BlockSpec((1, BLOCK_W), lambda j: (0, j)),
        ],
        out_specs=[
            pl.BlockSpec((N_ROWS, BLOCK_W), lambda j: (0, j)),
            pl.BlockSpec((1, BLOCK_W), lambda j: (0, j)),
        ],
        out_shape=[
            jax.ShapeDtypeStruct((N_ROWS, N_NODES), jnp.float32),
            jax.ShapeDtypeStruct((1, N_NODES), jnp.float32),
        ],
    )(t, buf, w2d)
    return (out_buf, nx2d.reshape(N_NODES))


# speedup vs baseline: 1.1184x; 1.0023x over previous
import functools

import jax
import jax.numpy as jnp
from jax.experimental import pallas as pl
from jax.experimental.pallas import tpu as pltpu

NH = 512
DT = 1.0
N_NODES = 100000
N_ROWS = NH + 2

BLOCK_W = 6144


def _step_kernel(t_ref, buf_ref, w_ref, out_ref, nx_ref):
    tt = t_ref[0, 0]
    out_ref[...] = buf_ref[...]
    x = buf_ref[pl.ds(NH + tt, 1), :]
    r0 = buf_ref[pl.ds(tt, 1), :]
    r1 = buf_ref[pl.ds(tt + 1, 1), :]
    w = w_ref[...]
    d1 = 0.1 * (r0 - x)
    xi = jnp.tanh(x + DT * d1 + w)
    d2 = 0.1 * (r1 - xi)
    nx = jnp.tanh(x + DT * 0.5 * (d1 + d2) + w)
    out_ref[pl.ds(NH + tt + 1, 1), :] = nx
    nx_ref[...] = nx


@functools.partial(jax.jit, static_argnames=())
def kernel(buf, dWt, t):
    w2d = dWt.reshape(1, N_NODES)
    grid = (pl.cdiv(N_NODES, BLOCK_W),)
    out_buf, nx2d = pl.pallas_call(
        _step_kernel,
        grid=grid,
        in_specs=[
            pl.BlockSpec(memory_space=pltpu.SMEM),
            pl.BlockSpec((N_ROWS, BLOCK_W), lambda j: (0, j)),
            pl.BlockSpec((1, BLOCK_W), lambda j: (0, j)),
        ],
        out_specs=[
            pl.BlockSpec((N_ROWS, BLOCK_W), lambda j: (0, j)),
            pl.BlockSpec((1, BLOCK_W), lambda j: (0, j)),
        ],
        out_shape=[
            jax.ShapeDtypeStruct((N_ROWS, N_NODES), jnp.float32),
            jax.ShapeDtypeStruct((1, N_NODES), jnp.float32),
        ],
    )(t, buf, w2d)
    return (out_buf, nx2d.reshape(N_NODES))
